# baseline (speedup 1.0000x reference)
import functools

import jax
import jax.numpy as jnp
from jax.experimental import pallas as pl
from jax.experimental.pallas import tpu as pltpu


def _ln_relu(t, g, beta):
    m = jnp.mean(t, axis=-1, keepdims=True)
    v = jnp.mean((t - m) ** 2, axis=-1, keepdims=True)
    h = (t - m) * jax.lax.rsqrt(v + 1e-5) * g + beta
    return jnp.maximum(h, 0.0)


def _fused_kernel(adj_a_ref, adj_b_ref, x_ref, w1_ref, b1_ref, g1_ref,
                  beta1_ref, w2_ref, b2_ref, g2_ref, beta2_ref, wc_ref,
                  bc_ref, out_ref, h1_ref, *, t_steps, bn):
    i = pl.program_id(0)
    hb = bn // 2

    def layer1(_):
        for k, a_ref in enumerate((adj_a_ref, adj_b_ref)):
            s = jnp.dot(a_ref[...], x_ref[...],
                        preferred_element_type=jnp.float32)
            t = jnp.dot(s, w1_ref[...].T,
                        preferred_element_type=jnp.float32) + b1_ref[...]
            h1_ref[pl.ds(i * bn + k * hb, hb), :] = _ln_relu(
                t, g1_ref[...], beta1_ref[...])
        return 0

    def layer2(_):
        for k, a_ref in enumerate((adj_a_ref, adj_b_ref)):
            s = jnp.dot(a_ref[...], h1_ref[...],
                        preferred_element_type=jnp.float32)
            t = jnp.dot(s, w2_ref[...].T,
                        preferred_element_type=jnp.float32) + b2_ref[...]
            h = _ln_relu(t, g2_ref[...], beta2_ref[...])
            out_ref[pl.ds(k * hb, hb), :] = (
                jnp.sum(h * wc_ref[...], axis=-1, keepdims=True) + bc_ref[0])
        return 0

    jax.lax.cond(i < t_steps, layer1, layer2, 0)


def _row_block(i, t_steps):
    return jnp.where(i < t_steps, i, 2 * t_steps - 1 - i)


def _pick_block(n: int) -> int:
    for bn in (400, 250, 200, 125, 100, 50, 25, 16, 8):
        if n % bn == 0:
            return bn
    return n


@functools.partial(jax.jit, static_argnames=())
def kernel(x, adj, W1, b1, g1, beta1, W2, b2, g2, beta2, Wc, bc):
    n, d = x.shape
    h_dim = W1.shape[0]
    bn = _pick_block(n)
    t_steps = n // bn
    hb = bn // 2

    full = lambda shape: pl.BlockSpec(shape, lambda i: (0,) * len(shape))

    out = pl.pallas_call(
        functools.partial(_fused_kernel, t_steps=t_steps, bn=bn),
        grid=(2 * t_steps,),
        in_specs=[
            pl.BlockSpec((hb, n),
                         lambda i, T=t_steps: (2 * _row_block(i, T), 0)),
            pl.BlockSpec((hb, n),
                         lambda i, T=t_steps: (2 * _row_block(i, T) + 1, 0)),
            full((n, d)),
            full(W1.shape),
            full(b1.shape),
            full(g1.shape),
            full(beta1.shape),
            full(W2.shape),
            full(b2.shape),
            full(g2.shape),
            full(beta2.shape),
            full(Wc.shape),
            full(bc.shape),
        ],
        out_specs=pl.BlockSpec(
            (bn, 1),
            lambda i, T=t_steps: (jnp.where(i < T, T - 1, 2 * T - 1 - i), 0)),
        out_shape=jax.ShapeDtypeStruct((n, 1), jnp.float32),
        scratch_shapes=[pltpu.VMEM((n, h_dim), jnp.float32)],
        compiler_params=pltpu.CompilerParams(
            dimension_semantics=("arbitrary",),
            vmem_limit_bytes=64 * 1024 * 1024,
        ),
    )(adj, adj, x, W1, b1, g1, beta1, W2, b2, g2, beta2, Wc, bc)

    return out.reshape(n)


# speedup vs baseline: 1.1099x; 1.1099x over previous
import functools

import jax
import jax.numpy as jnp
from jax.experimental import pallas as pl
from jax.experimental.pallas import tpu as pltpu


def _ln_relu(t, g, beta):
    m = jnp.mean(t, axis=-1, keepdims=True)
    v = jnp.mean((t - m) ** 2, axis=-1, keepdims=True)
    h = (t - m) * jax.lax.rsqrt(v + 1e-5) * g + beta
    return jnp.maximum(h, 0.0)


def _fused_kernel(adj_ref, x_ref, w1_ref, b1_ref, g1_ref, beta1_ref,
                  w2_ref, b2_ref, g2_ref, beta2_ref, wc_ref, bc_ref,
                  out_ref, h1_ref, *, t_steps, bn):
    i = pl.program_id(0)

    def layer1(_):
        s = jnp.dot(adj_ref[...], x_ref[...],
                    preferred_element_type=jnp.float32)
        t = jnp.dot(s, w1_ref[...].T,
                    preferred_element_type=jnp.float32) + b1_ref[...]
        h1_ref[pl.ds(i * bn, bn), :] = _ln_relu(t, g1_ref[...], beta1_ref[...])
        return 0

    def layer2(_):
        s = jnp.dot(adj_ref[...], h1_ref[...],
                    preferred_element_type=jnp.float32)
        t = jnp.dot(s, w2_ref[...].T,
                    preferred_element_type=jnp.float32) + b2_ref[...]
        h = _ln_relu(t, g2_ref[...], beta2_ref[...])
        out_ref[...] = (jnp.sum(h * wc_ref[...], axis=-1, keepdims=True)
                        + bc_ref[0])
        return 0

    jax.lax.cond(i < t_steps, layer1, layer2, 0)


def _row_block(i, t_steps):
    return jnp.where(i < t_steps, i, 2 * t_steps - 1 - i)


def _pick_block(n: int) -> int:
    for bn in (400, 250, 200, 125, 100, 50, 25, 16, 8):
        if n % bn == 0:
            return bn
    return n


@functools.partial(jax.jit, static_argnames=())
def kernel(x, adj, W1, b1, g1, beta1, W2, b2, g2, beta2, Wc, bc):
    n, d = x.shape
    h_dim = W1.shape[0]
    bn = _pick_block(n)
    t_steps = n // bn

    full = lambda shape: pl.BlockSpec(shape, lambda i: (0,) * len(shape))

    out = pl.pallas_call(
        functools.partial(_fused_kernel, t_steps=t_steps, bn=bn),
        grid=(2 * t_steps,),
        in_specs=[
            pl.BlockSpec((bn, n), lambda i, T=t_steps: (_row_block(i, T), 0)),
            full((n, d)),
            full(W1.shape),
            full(b1.shape),
            full(g1.shape),
            full(beta1.shape),
            full(W2.shape),
            full(b2.shape),
            full(g2.shape),
            full(beta2.shape),
            full(Wc.shape),
            full(bc.shape),
        ],
        out_specs=pl.BlockSpec(
            (bn, 1),
            lambda i, T=t_steps: (jnp.where(i < T, T - 1, 2 * T - 1 - i), 0)),
        out_shape=jax.ShapeDtypeStruct((n, 1), jnp.float32),
        scratch_shapes=[pltpu.VMEM((n, h_dim), jnp.float32)],
        compiler_params=pltpu.CompilerParams(
            dimension_semantics=("arbitrary",),
            vmem_limit_bytes=64 * 1024 * 1024,
        ),
    )(adj, x, W1, b1, g1, beta1, W2, b2, g2, beta2, Wc, bc)

    return out.reshape(n)
